# Initial kernel scaffold; baseline (speedup 1.0000x reference)
import jax
import jax.numpy as jnp
from jax.experimental import pallas as pl


def kernel(x, w, b):
    raise NotImplementedError("write your pallas kernel here")


# speedup vs baseline: 2.9345x; 2.9345x over previous
import functools

import jax
import jax.numpy as jnp
from jax import lax
from jax.experimental import pallas as pl
from jax.experimental.pallas import tpu as pltpu


def _conv_body(xs_ref, w_ref, b_ref, o_ref, *, tm, ext, row_stride, kh, kw):
    mi = pl.program_id(1)
    q0 = pl.multiple_of(mi * tm, tm)
    base = xs_ref[0, :, pl.ds(q0, tm + ext)]
    parts = [
        base[:, dh * row_stride + dw:dh * row_stride + dw + tm]
        for dh in range(kh)
        for dw in range(kw)
    ]
    xk = jnp.concatenate(parts, axis=0)
    acc = lax.dot_general(
        w_ref[...], xk, (((1,), (0,)), ((), ())),
        preferred_element_type=jnp.float32)
    o_ref[0] = (acc + b_ref[:, :1]).astype(o_ref.dtype)


@jax.jit
def _conv2d(x, w, b):
    C_out, C_in, kh, kw = w.shape
    B, _, H, W = x.shape
    Ho = H - kh + 1
    Wo = W - kw + 1
    P = H * W

    TM = 512 if P % 512 == 0 else P
    n_m = P // TM
    off_max = (kh - 1) * W + (kw - 1)
    ext = pl.cdiv(off_max + 1, 128) * 128
    P_pad = P + ext

    xs = jnp.pad(x.reshape(B, C_in, P).astype(jnp.bfloat16),
                 ((0, 0), (0, 0), (0, P_pad - P)))
    wp = w.transpose(0, 2, 3, 1).reshape(C_out, kh * kw * C_in)
    wp = wp.astype(jnp.bfloat16)
    bb = jnp.broadcast_to(b.astype(jnp.float32).reshape(C_out, 1),
                          (C_out, 128))

    body = functools.partial(_conv_body, tm=TM, ext=ext, row_stride=W,
                             kh=kh, kw=kw)
    y = pl.pallas_call(
        body,
        out_shape=jax.ShapeDtypeStruct((B, C_out, P), jnp.bfloat16),
        grid=(B, n_m),
        in_specs=[
            pl.BlockSpec((1, C_in, P_pad), lambda bi, mi: (bi, 0, 0)),
            pl.BlockSpec((C_out, kh * kw * C_in), lambda bi, mi: (0, 0)),
            pl.BlockSpec((C_out, 128), lambda bi, mi: (0, 0)),
        ],
        out_specs=pl.BlockSpec((1, C_out, TM), lambda bi, mi: (bi, 0, mi)),
        compiler_params=pltpu.CompilerParams(
            dimension_semantics=("parallel", "arbitrary"),
            vmem_limit_bytes=int(48 << 20)),
    )(xs, wp, bb)

    return (y.reshape(B, C_out, H, W)[:, :, :Ho, :Wo]
            .astype(jnp.float32))


def kernel(x, w, b):
    return _conv2d(x, w, b)


# speedup vs baseline: 3.6000x; 1.2268x over previous
import functools

import jax
import jax.numpy as jnp
from jax import lax
from jax.experimental import pallas as pl
from jax.experimental.pallas import tpu as pltpu


def _conv_body(xs_ref, w_ref, b_ref, o_ref, *, tm, ext, row_stride, kh, kw):
    mi = pl.program_id(1)
    q0 = pl.multiple_of(mi * tm, tm)
    base = xs_ref[0, :, pl.ds(q0, tm + ext)]
    parts = [
        base[:, dh * row_stride + dw:dh * row_stride + dw + tm]
        for dh in range(kh)
        for dw in range(kw)
    ]
    xk = jnp.concatenate(parts, axis=0)
    acc = lax.dot_general(
        w_ref[...], xk, (((1,), (0,)), ((), ())),
        preferred_element_type=jnp.float32)
    o_ref[0] = (acc + b_ref[:, :1]).astype(o_ref.dtype)


@jax.jit
def _conv2d(x, w, b):
    C_out, C_in, kh, kw = w.shape
    B, _, H, W = x.shape
    Ho = H - kh + 1
    Wo = W - kw + 1
    P = H * W

    TM = 1024 if P % 1024 == 0 else P
    n_m = P // TM
    off_max = (kh - 1) * W + (kw - 1)
    ext = pl.cdiv(off_max + 1, 128) * 128
    P_pad = P + ext

    xs = jnp.pad(x.reshape(B, C_in, P).astype(jnp.bfloat16),
                 ((0, 0), (0, 0), (0, P_pad - P)))
    wp = w.transpose(0, 2, 3, 1).reshape(C_out, kh * kw * C_in)
    wp = wp.astype(jnp.bfloat16)
    bb = jnp.broadcast_to(b.astype(jnp.float32).reshape(C_out, 1),
                          (C_out, 128))

    body = functools.partial(_conv_body, tm=TM, ext=ext, row_stride=W,
                             kh=kh, kw=kw)
    y = pl.pallas_call(
        body,
        out_shape=jax.ShapeDtypeStruct((B, C_out, P), jnp.bfloat16),
        grid=(B, n_m),
        in_specs=[
            pl.BlockSpec((1, C_in, P_pad), lambda bi, mi: (bi, 0, 0)),
            pl.BlockSpec((C_out, kh * kw * C_in), lambda bi, mi: (0, 0)),
            pl.BlockSpec((C_out, 128), lambda bi, mi: (0, 0)),
        ],
        out_specs=pl.BlockSpec((1, C_out, TM), lambda bi, mi: (bi, 0, mi)),
        compiler_params=pltpu.CompilerParams(
            dimension_semantics=("parallel", "arbitrary"),
            vmem_limit_bytes=int(48 << 20)),
    )(xs, wp, bb)

    return (y.reshape(B, C_out, H, W)[:, :, :Ho, :Wo]
            .astype(jnp.float32))


def kernel(x, w, b):
    return _conv2d(x, w, b)


# speedup vs baseline: 4.0896x; 1.1360x over previous
import functools

import jax
import jax.numpy as jnp
from jax import lax
from jax.experimental import pallas as pl
from jax.experimental.pallas import tpu as pltpu


def _conv_body(xs_ref, w_ref, b_ref, o_ref, *, tm, ext, row_stride, kh, kw):
    mi = pl.program_id(1)
    q0 = pl.multiple_of(mi * tm, tm)
    base = xs_ref[0, :, pl.ds(q0, tm + ext)]
    parts = [
        base[:, dh * row_stride + dw:dh * row_stride + dw + tm]
        for dh in range(kh)
        for dw in range(kw)
    ]
    xk = jnp.concatenate(parts, axis=0)
    acc = lax.dot_general(
        w_ref[...], xk, (((1,), (0,)), ((), ())),
        preferred_element_type=jnp.float32)
    o_ref[0] = (acc + b_ref[:, :1]).astype(o_ref.dtype)


@jax.jit
def _conv2d(x, w, b):
    C_out, C_in, kh, kw = w.shape
    B, _, H, W = x.shape
    Ho = H - kh + 1
    Wo = W - kw + 1
    P = H * W

    TM = 2048 if P % 2048 == 0 else P
    n_m = P // TM
    off_max = (kh - 1) * W + (kw - 1)
    ext = pl.cdiv(off_max + 1, 128) * 128
    P_pad = P + ext

    xs = jnp.pad(x.reshape(B, C_in, P).astype(jnp.bfloat16),
                 ((0, 0), (0, 0), (0, P_pad - P)))
    wp = w.transpose(0, 2, 3, 1).reshape(C_out, kh * kw * C_in)
    wp = wp.astype(jnp.bfloat16)
    bb = jnp.broadcast_to(b.astype(jnp.float32).reshape(C_out, 1),
                          (C_out, 128))

    body = functools.partial(_conv_body, tm=TM, ext=ext, row_stride=W,
                             kh=kh, kw=kw)
    y = pl.pallas_call(
        body,
        out_shape=jax.ShapeDtypeStruct((B, C_out, P), jnp.bfloat16),
        grid=(B, n_m),
        in_specs=[
            pl.BlockSpec((1, C_in, P_pad), lambda bi, mi: (bi, 0, 0)),
            pl.BlockSpec((C_out, kh * kw * C_in), lambda bi, mi: (0, 0)),
            pl.BlockSpec((C_out, 128), lambda bi, mi: (0, 0)),
        ],
        out_specs=pl.BlockSpec((1, C_out, TM), lambda bi, mi: (bi, 0, mi)),
        compiler_params=pltpu.CompilerParams(
            dimension_semantics=("parallel", "arbitrary"),
            vmem_limit_bytes=int(48 << 20)),
    )(xs, wp, bb)

    return (y.reshape(B, C_out, H, W)[:, :, :Ho, :Wo]
            .astype(jnp.float32))


def kernel(x, w, b):
    return _conv2d(x, w, b)


# speedup vs baseline: 4.5445x; 1.1112x over previous
import functools

import jax
import jax.numpy as jnp
from jax import lax
from jax.experimental import pallas as pl
from jax.experimental.pallas import tpu as pltpu


def _conv_body(xs_ref, w_ref, b_ref, o_ref, *, tm, ext, row_stride, kh, kw):
    mi = pl.program_id(1)
    q0 = pl.multiple_of(mi * tm, tm)
    base = xs_ref[0, :, pl.ds(q0, tm + ext)]
    parts = [
        base[:, dh * row_stride + dw:dh * row_stride + dw + tm]
        for dh in range(kh)
        for dw in range(kw)
    ]
    xk = jnp.concatenate(parts, axis=0)
    acc = lax.dot_general(
        w_ref[...], xk, (((1,), (0,)), ((), ())),
        preferred_element_type=jnp.float32)
    o_ref[0] = (acc + b_ref[:, :1]).astype(o_ref.dtype)


@jax.jit
def _conv2d(x, w, b):
    C_out, C_in, kh, kw = w.shape
    B, _, H, W = x.shape
    Ho = H - kh + 1
    Wo = W - kw + 1
    P = H * W

    TM = P
    n_m = P // TM
    off_max = (kh - 1) * W + (kw - 1)
    ext = pl.cdiv(off_max + 1, 128) * 128
    P_pad = P + ext

    xs = jnp.pad(x.reshape(B, C_in, P).astype(jnp.bfloat16),
                 ((0, 0), (0, 0), (0, P_pad - P)))
    wp = w.transpose(0, 2, 3, 1).reshape(C_out, kh * kw * C_in)
    wp = wp.astype(jnp.bfloat16)
    bb = jnp.broadcast_to(b.astype(jnp.float32).reshape(C_out, 1),
                          (C_out, 128))

    body = functools.partial(_conv_body, tm=TM, ext=ext, row_stride=W,
                             kh=kh, kw=kw)
    y = pl.pallas_call(
        body,
        out_shape=jax.ShapeDtypeStruct((B, C_out, P), jnp.bfloat16),
        grid=(B, n_m),
        in_specs=[
            pl.BlockSpec((1, C_in, P_pad), lambda bi, mi: (bi, 0, 0)),
            pl.BlockSpec((C_out, kh * kw * C_in), lambda bi, mi: (0, 0)),
            pl.BlockSpec((C_out, 128), lambda bi, mi: (0, 0)),
        ],
        out_specs=pl.BlockSpec((1, C_out, TM), lambda bi, mi: (bi, 0, mi)),
        compiler_params=pltpu.CompilerParams(
            dimension_semantics=("parallel", "arbitrary"),
            vmem_limit_bytes=int(48 << 20)),
    )(xs, wp, bb)

    return (y.reshape(B, C_out, H, W)[:, :, :Ho, :Wo]
            .astype(jnp.float32))


def kernel(x, w, b):
    return _conv2d(x, w, b)


# speedup vs baseline: 4.6990x; 1.0340x over previous
import functools

import jax
import jax.numpy as jnp
from jax import lax
from jax.experimental import pallas as pl
from jax.experimental.pallas import tpu as pltpu


def _conv_body(xt_ref, w_ref, b_ref, o_ref, *, H, W, kh, kw, n_ext):
    pieces = [xt_ref[0, min(h, H - 1)] for h in range(H + n_ext)]
    slab = jnp.concatenate(pieces, axis=1)
    parts = [
        slab[:, dh * W + dw:dh * W + dw + H * W]
        for dh in range(kh)
        for dw in range(kw)
    ]
    xk = jnp.concatenate(parts, axis=0)
    acc = lax.dot_general(
        w_ref[...], xk, (((1,), (0,)), ((), ())),
        preferred_element_type=jnp.float32)
    o_ref[0] = (acc + b_ref[:, :1]).astype(o_ref.dtype)


@jax.jit
def _conv2d(x, w, b):
    C_out, C_in, kh, kw = w.shape
    B, _, H, W = x.shape
    Ho = H - kh + 1
    Wo = W - kw + 1
    P = H * W
    n_ext = kh

    xt = x.transpose(0, 2, 1, 3).astype(jnp.bfloat16)
    wp = w.transpose(0, 2, 3, 1).reshape(C_out, kh * kw * C_in)
    wp = wp.astype(jnp.bfloat16)
    bb = jnp.broadcast_to(b.astype(jnp.float32).reshape(C_out, 1),
                          (C_out, 128))

    body = functools.partial(_conv_body, H=H, W=W, kh=kh, kw=kw, n_ext=n_ext)
    y = pl.pallas_call(
        body,
        out_shape=jax.ShapeDtypeStruct((B, C_out, P), jnp.bfloat16),
        grid=(B,),
        in_specs=[
            pl.BlockSpec((1, H, C_in, W), lambda bi: (bi, 0, 0, 0)),
            pl.BlockSpec((C_out, kh * kw * C_in), lambda bi: (0, 0)),
            pl.BlockSpec((C_out, 128), lambda bi: (0, 0)),
        ],
        out_specs=pl.BlockSpec((1, C_out, P), lambda bi: (bi, 0, 0)),
        compiler_params=pltpu.CompilerParams(
            dimension_semantics=("parallel",),
            vmem_limit_bytes=int(48 << 20)),
    )(xt, wp, bb)

    return (y.reshape(B, C_out, H, W)[:, :, :Ho, :Wo]
            .astype(jnp.float32))


def kernel(x, w, b):
    return _conv2d(x, w, b)
